# Initial kernel scaffold; baseline (speedup 1.0000x reference)
import jax
import jax.numpy as jnp
from jax.experimental import pallas as pl


def kernel(z, gamma, beta):
    raise NotImplementedError("write your pallas kernel here")


# speedup vs baseline: 18.3214x; 18.3214x over previous
import functools

import jax
import jax.numpy as jnp
from jax.experimental import pallas as pl
from jax.experimental.pallas import tpu as pltpu

_B = 32
_LANES = 128
_TS = 256


def _cbn_kernel(x_ref, par_ref, o_ref):
    x = x_ref[...]
    ts = x.shape[1]
    inv_b = jnp.float32(1.0 / _B)
    inv_bm1 = jnp.float32(1.0 / (_B - 1))

    s1 = jnp.sum(x, axis=0)
    s2 = jnp.sum(x * x, axis=0)
    xs = jnp.roll(x, -1, axis=-1)
    sx = jnp.sum(x * xs, axis=0)

    lane = jax.lax.broadcasted_iota(jnp.int32, (ts, _LANES), 1)
    even = (lane & 1) == 0

    mu = s1 * inv_b
    mu_n = jnp.roll(mu, -1, axis=-1)
    var = (s2 - (_B * mu) * mu) * inv_bm1
    var_n = jnp.roll(var, -1, axis=-1)
    sri = (sx - (_B * mu) * mu_n) * inv_bm1

    tr = var + var_n
    det = var * var_n - sri * sri
    s = jnp.sqrt(det)
    it = jax.lax.rsqrt(tr + 2.0 * s)
    m00 = (var + s) * it
    m11 = (var_n + s) * it
    moff = sri * it

    diag = jnp.where(even, m00, jnp.roll(m11, 1, axis=-1))
    offf = jnp.where(even, moff, jnp.roll(moff, 1, axis=-1))
    anti = jnp.where(even, m11, jnp.roll(m00, 1, axis=-1))

    a_d = par_ref[0:1, :]
    a_o = par_ref[1:2, :]
    bv = par_ref[2:3, :]
    wd = a_d * diag + a_o * offf
    wo = a_d * offf + a_o * anti
    pmu = jnp.where(even, mu_n, jnp.roll(mu, 1, axis=-1))
    bc = bv - wd * mu - wo * pmu

    p = jnp.where(even, xs, jnp.roll(x, 1, axis=-1))
    o_ref[...] = wd[None] * x + (wo[None] * p + bc[None])


@jax.jit
def kernel(z, gamma, beta):
    B, C, H, W, _ = z.shape
    S = C * H * W * 2 // _LANES
    xv = z.reshape(B, S, _LANES)

    lane = jnp.arange(_LANES) % 2
    a_d = jnp.where(lane == 0, gamma[0, 0], gamma[1, 1])
    a_o = jnp.where(lane == 0, gamma[0, 1], gamma[1, 0])
    bv = jnp.where(lane == 0, beta[0], beta[1])
    params = jnp.concatenate(
        [a_d[None], a_o[None], bv[None], jnp.zeros((5, _LANES), jnp.float32)],
        axis=0)

    grid = (S // _TS,)
    out = pl.pallas_call(
        _cbn_kernel,
        grid=grid,
        in_specs=[
            pl.BlockSpec((B, _TS, _LANES), lambda i: (0, i, 0)),
            pl.BlockSpec((8, _LANES), lambda i: (0, 0)),
        ],
        out_specs=pl.BlockSpec((B, _TS, _LANES), lambda i: (0, i, 0)),
        out_shape=jax.ShapeDtypeStruct((B, S, _LANES), jnp.float32),
        compiler_params=pltpu.CompilerParams(
            dimension_semantics=("parallel",),
            vmem_limit_bytes=56 * 1024 * 1024,
        ),
    )(xv, params)
    return out.reshape(B, C, H, W, 2)


# speedup vs baseline: 188.6312x; 10.2957x over previous
import jax
import jax.numpy as jnp
from jax.experimental import pallas as pl
from jax.experimental.pallas import tpu as pltpu

_B = 32
_LANES = 128
_TS = 512


def _cbn_kernel(x_ref, par_ref, o_ref):
    xr = x_ref[:, ::2, :]
    xi = x_ref[:, 1::2, :]
    inv_b = jnp.float32(1.0 / _B)
    inv_bm1 = jnp.float32(1.0 / (_B - 1))

    mur = jnp.sum(xr, axis=0) * inv_b
    mui = jnp.sum(xi, axis=0) * inv_b
    srr = (jnp.sum(xr * xr, axis=0) - (_B * mur) * mur) * inv_bm1
    sii = (jnp.sum(xi * xi, axis=0) - (_B * mui) * mui) * inv_bm1
    sri = (jnp.sum(xr * xi, axis=0) - (_B * mur) * mui) * inv_bm1

    s = jnp.sqrt(srr * sii - sri * sri)
    it = jax.lax.rsqrt(srr + sii + 2.0 * s)
    m00 = (srr + s) * it
    m11 = (sii + s) * it
    moff = sri * it

    g00 = par_ref[0:1, :]
    g01 = par_ref[1:2, :]
    g10 = par_ref[2:3, :]
    g11 = par_ref[3:4, :]
    w00 = g00 * m00 + g01 * moff
    w01 = g00 * moff + g01 * m11
    w10 = g10 * m00 + g11 * moff
    w11 = g10 * moff + g11 * m11
    bc0 = par_ref[4:5, :] - w00 * mur - w01 * mui
    bc1 = par_ref[5:6, :] - w10 * mur - w11 * mui

    o_ref[:, ::2, :] = w00[None] * xr + (w01[None] * xi + bc0[None])
    o_ref[:, 1::2, :] = w10[None] * xr + (w11[None] * xi + bc1[None])


@jax.jit
def kernel(z, gamma, beta):
    B, C, H, W, _ = z.shape
    S = H * W * 2
    xv = z.transpose(0, 2, 3, 4, 1).reshape(B, S, _LANES)

    ones = jnp.ones((_LANES,), jnp.float32)
    params = jnp.stack([
        gamma[0, 0] * ones, gamma[0, 1] * ones,
        gamma[1, 0] * ones, gamma[1, 1] * ones,
        beta[0] * ones, beta[1] * ones,
        jnp.zeros((_LANES,), jnp.float32), jnp.zeros((_LANES,), jnp.float32),
    ], axis=0)

    grid = (S // _TS,)
    out = pl.pallas_call(
        _cbn_kernel,
        grid=grid,
        in_specs=[
            pl.BlockSpec((B, _TS, _LANES), lambda i: (0, i, 0)),
            pl.BlockSpec((8, _LANES), lambda i: (0, 0)),
        ],
        out_specs=pl.BlockSpec((B, _TS, _LANES), lambda i: (0, i, 0)),
        out_shape=jax.ShapeDtypeStruct((B, S, _LANES), jnp.float32),
        compiler_params=pltpu.CompilerParams(
            dimension_semantics=("parallel",),
            vmem_limit_bytes=56 * 1024 * 1024,
        ),
    )(xv, params)
    return out.reshape(B, H, W, 2, C).transpose(0, 4, 1, 2, 3)


# speedup vs baseline: 188.6693x; 1.0002x over previous
import jax
import jax.numpy as jnp
from jax.experimental import pallas as pl
from jax.experimental.pallas import tpu as pltpu

_B = 32
_LANES = 128
_TS = 512


def _cbn_kernel(x_ref, par_ref, o_ref):
    xr = x_ref[:, ::2, :]
    xi = x_ref[:, 1::2, :]
    inv_b = jnp.float32(1.0 / _B)
    inv_bm1 = jnp.float32(1.0 / (_B - 1))

    mur = jnp.sum(xr, axis=0) * inv_b
    mui = jnp.sum(xi, axis=0) * inv_b
    srr = (jnp.sum(xr * xr, axis=0) - (_B * mur) * mur) * inv_bm1
    sii = (jnp.sum(xi * xi, axis=0) - (_B * mui) * mui) * inv_bm1
    sri = (jnp.sum(xr * xi, axis=0) - (_B * mur) * mui) * inv_bm1

    s = jnp.sqrt(srr * sii - sri * sri)
    it = jax.lax.rsqrt(srr + sii + 2.0 * s)
    m00 = (srr + s) * it
    m11 = (sii + s) * it
    moff = sri * it

    g00 = par_ref[0:1, :]
    g01 = par_ref[1:2, :]
    g10 = par_ref[2:3, :]
    g11 = par_ref[3:4, :]
    w00 = g00 * m00 + g01 * moff
    w01 = g00 * moff + g01 * m11
    w10 = g10 * m00 + g11 * moff
    w11 = g10 * moff + g11 * m11
    bc0 = par_ref[4:5, :] - w00 * mur - w01 * mui
    bc1 = par_ref[5:6, :] - w10 * mur - w11 * mui

    o_ref[:, ::2, :] = w00[None] * xr + (w01[None] * xi + bc0[None])
    o_ref[:, 1::2, :] = w10[None] * xr + (w11[None] * xi + bc1[None])


@jax.jit
def kernel(z, gamma, beta):
    B, C, H, W, _ = z.shape
    S = H * W * 2
    xv = z.transpose(0, 2, 3, 4, 1).reshape(B, S, _LANES)

    ones = jnp.ones((_LANES,), jnp.float32)
    params = jnp.stack([
        gamma[0, 0] * ones, gamma[0, 1] * ones,
        gamma[1, 0] * ones, gamma[1, 1] * ones,
        beta[0] * ones, beta[1] * ones,
        jnp.zeros((_LANES,), jnp.float32), jnp.zeros((_LANES,), jnp.float32),
    ], axis=0)

    grid = (S // _TS,)
    out = pl.pallas_call(
        _cbn_kernel,
        grid=grid,
        in_specs=[
            pl.BlockSpec((B, _TS, _LANES), lambda i: (0, i, 0)),
            pl.BlockSpec((8, _LANES), lambda i: (0, 0)),
        ],
        out_specs=pl.BlockSpec((B, _TS, _LANES), lambda i: (0, i, 0)),
        out_shape=jax.ShapeDtypeStruct((B, S, _LANES), jnp.float32),
        compiler_params=pltpu.CompilerParams(
            dimension_semantics=("arbitrary",),
            vmem_limit_bytes=56 * 1024 * 1024,
        ),
    )(xv, params)
    return out.reshape(B, H, W, 2, C).transpose(0, 4, 1, 2, 3)
